# baseline (speedup 1.0000x reference)
import functools

import jax
import jax.numpy as jnp
from jax import lax
from jax.experimental import pallas as pl
from jax.experimental.pallas import tpu as pltpu
from jax.experimental.pallas import tpu_sc as plsc

_LOG_2PI = 1.8378770664093453


def _sc_gather(times, mean_params, std_params, b, d):
    info = plsc.get_sparse_core_info()
    nw = info.num_cores * info.num_subcores
    bpw = b // nw
    chunk = 128
    n_chunks = bpw // chunk

    mesh = plsc.VectorSubcoreMesh(core_axis_name="c", subcore_axis_name="s")

    @functools.partial(
        pl.kernel,
        mesh=mesh,
        out_type=[
            jax.ShapeDtypeStruct((b, d), jnp.float32),
            jax.ShapeDtypeStruct((b, d), jnp.float32),
        ],
        scratch_types=[
            pltpu.VMEM((n_chunks, chunk), jnp.int32),
            pltpu.VMEM((bpw, d), jnp.float32),
            pltpu.VMEM((bpw, d), jnp.float32),
            pltpu.SemaphoreType.DMA,
        ],
        compiler_params=pltpu.CompilerParams(use_tc_tiling_on_sc=False),
    )
    def run(times_hbm, mean_hbm, std_hbm, mu_out, sd_out,
            idx_v, mu_v, sd_v, sem):
        wid = lax.axis_index("s") * info.num_cores + lax.axis_index("c")
        base = wid * bpw
        for c in range(n_chunks):
            pltpu.sync_copy(times_hbm.at[pl.ds(base + c * chunk, chunk)],
                            idx_v.at[c])
        copies = []
        for c in range(n_chunks):
            row_sl = pl.ds(c * chunk, chunk)
            copies.append(pltpu.async_copy(
                mean_hbm.at[idx_v.at[c]], mu_v.at[row_sl], sem))
            copies.append(pltpu.async_copy(
                std_hbm.at[idx_v.at[c]], sd_v.at[row_sl], sem))
        for cp in copies:
            cp.wait()
        pltpu.sync_copy(mu_v, mu_out.at[pl.ds(base, bpw)])
        pltpu.sync_copy(sd_v, sd_out.at[pl.ds(base, bpw)])

    return run(times, mean_params, std_params)


def _tc_log_prob(states, mu, sd, b, d):
    blk = 1024
    grid = b // blk

    def body(st_ref, mu_ref, sd_ref, out_ref):
        s = st_ref[...]
        m = mu_ref[...]
        sig = jnp.maximum(sd_ref[...], 0.01)
        t = (s - m) / sig
        lp = -0.5 * (t * t) - jnp.log(sig)
        out_ref[0, 0, :] = jnp.sum(lp, axis=-1) - (d * 0.5) * _LOG_2PI

    out = pl.pallas_call(
        body,
        grid=(grid,),
        in_specs=[
            pl.BlockSpec((blk, d), lambda i: (i, 0)),
            pl.BlockSpec((blk, d), lambda i: (i, 0)),
            pl.BlockSpec((blk, d), lambda i: (i, 0)),
        ],
        out_specs=pl.BlockSpec((1, 1, blk), lambda i: (i, 0, 0)),
        out_shape=jax.ShapeDtypeStruct((grid, 1, blk), jnp.float32),
    )(states, mu, sd)
    return out.reshape(b)


def kernel(states, times, mean_params, std_params):
    b, d = states.shape
    times = times.reshape(-1).astype(jnp.int32)
    mu, sd = _sc_gather(times, mean_params, std_params, b, d)
    return _tc_log_prob(states, mu, sd, b, d)


# speedup vs baseline: 1.0385x; 1.0385x over previous
import functools

import jax
import jax.numpy as jnp
from jax import lax
from jax.experimental import pallas as pl
from jax.experimental.pallas import tpu as pltpu
from jax.experimental.pallas import tpu_sc as plsc

_LN2 = 0.6931471805599453
_LOG_2PI = 1.8378770664093453
_SQRT2 = 1.4142135623730951
_CHUNK = 2048


def _log_sd(sd):
    bits = lax.bitcast_convert_type(sd, jnp.int32)
    e = (bits >> 23) - 127
    mant = lax.bitcast_convert_type(
        (bits & 0x007FFFFF) | 0x3F800000, jnp.float32)
    big = mant > _SQRT2
    mant = jnp.where(big, 0.5 * mant, mant)
    e = jnp.where(big, e + 1, e)
    w = (mant - 1.0) / (mant + 1.0)
    t2 = w * w
    poly = 1.0 + t2 * (0.33333333 + t2 * (0.2 + t2 * 0.14285715))
    return e.astype(jnp.float32) * _LN2 + 2.0 * w * poly


def _sc_partials(times, states_t, mean_t, std_t, b, d, n_times):
    info = plsc.get_sparse_core_info()
    nw = info.num_cores * info.num_subcores
    fpw = d // nw
    n_chunks = b // _CHUNK
    vregs = _CHUNK // 16

    mesh = plsc.VectorSubcoreMesh(core_axis_name="c", subcore_axis_name="s")

    @functools.partial(
        pl.kernel,
        mesh=mesh,
        out_type=jax.ShapeDtypeStruct((d, b), jnp.float32),
        scratch_types=[
            pltpu.VMEM((n_times,), jnp.float32),
            pltpu.VMEM((b,), jnp.float32),
            pltpu.VMEM((_CHUNK,), jnp.int32),
            pltpu.VMEM((_CHUNK,), jnp.float32),
        ],
        compiler_params=pltpu.CompilerParams(
            use_tc_tiling_on_sc=True,
            needs_layout_passes=False,
        ),
    )
    def run(times_hbm, states_hbm, mean_hbm, std_hbm, part_out,
            col_v, m_v, idx_v, st_v):
        wid = lax.axis_index("s") * info.num_cores + lax.axis_index("c")
        for fi in range(fpw):
            f = wid * fpw + fi
            pltpu.sync_copy(mean_hbm.at[f], col_v)

            def body_a(c, carry):
                pltpu.sync_copy(times_hbm.at[pl.ds(c * _CHUNK, _CHUNK)],
                                idx_v)

                def inner_a(v, carry2):
                    iv = idx_v[pl.ds(v * 16, 16)]
                    m_v[pl.ds(c * _CHUNK + v * 16, 16)] = (
                        plsc.load_gather(col_v, [iv]))
                    return carry2

                lax.fori_loop(0, vregs, inner_a, 0)
                return carry

            lax.fori_loop(0, n_chunks, body_a, 0)

            pltpu.sync_copy(std_hbm.at[f], col_v)

            def body_b(c, carry):
                pltpu.sync_copy(times_hbm.at[pl.ds(c * _CHUNK, _CHUNK)],
                                idx_v)
                pltpu.sync_copy(states_hbm.at[f, pl.ds(c * _CHUNK, _CHUNK)],
                                st_v)

                def inner_b(v, carry2):
                    iv = idx_v[pl.ds(v * 16, 16)]
                    sd = jnp.maximum(plsc.load_gather(col_v, [iv]), 0.01)
                    sl = pl.ds(c * _CHUNK + v * 16, 16)
                    m = m_v[sl]
                    s = st_v[pl.ds(v * 16, 16)]
                    t = (s - m) / sd
                    m_v[sl] = -0.5 * (t * t) - _log_sd(sd) - 0.5 * _LOG_2PI
                    return carry2

                lax.fori_loop(0, vregs, inner_b, 0)
                return carry

            lax.fori_loop(0, n_chunks, body_b, 0)
            pltpu.sync_copy(m_v, part_out.at[f])

    return run(times, states_t, mean_t, std_t)


def _tc_sum(part, b, d):
    blk = 1024
    grid = b // blk

    def body(p_ref, out_ref):
        out_ref[0, 0, :] = jnp.sum(p_ref[...], axis=0)

    out = pl.pallas_call(
        body,
        grid=(grid,),
        in_specs=[pl.BlockSpec((d, blk), lambda i: (0, i))],
        out_specs=pl.BlockSpec((1, 1, blk), lambda i: (i, 0, 0)),
        out_shape=jax.ShapeDtypeStruct((grid, 1, blk), jnp.float32),
    )(part)
    return out.reshape(b)


def kernel(states, times, mean_params, std_params):
    b, d = states.shape
    n_times = mean_params.shape[0]
    times = times.reshape(-1).astype(jnp.int32)
    part = _sc_partials(times, states.T, mean_params.T, std_params.T,
                        b, d, n_times)
    return _tc_sum(part, b, d)


# speedup vs baseline: 2.6075x; 2.5108x over previous
import functools

import jax
import jax.numpy as jnp
from jax import lax
from jax.experimental import pallas as pl
from jax.experimental.pallas import tpu as pltpu
from jax.experimental.pallas import tpu_sc as plsc

_LOG_2PI = 1.8378770664093453


def _sc_gather(times, mean_t, std_t, b, d, n_times):
    info = plsc.get_sparse_core_info()
    nw = info.num_cores * info.num_subcores
    fpw = d // nw
    half = b // 2

    mesh = plsc.VectorSubcoreMesh(core_axis_name="c", subcore_axis_name="s")

    @functools.partial(
        pl.kernel,
        mesh=mesh,
        out_type=[
            jax.ShapeDtypeStruct((d, b), jnp.float32),
            jax.ShapeDtypeStruct((d, b), jnp.float32),
        ],
        scratch_types=[
            pltpu.VMEM((n_times,), jnp.float32),
            pltpu.VMEM((half,), jnp.int32),
            pltpu.VMEM((b,), jnp.float32),
        ],
        compiler_params=pltpu.CompilerParams(
            use_tc_tiling_on_sc=True,
            needs_layout_passes=False,
        ),
    )
    def run(times_hbm, mean_hbm, std_hbm, mu_out, sd_out,
            col_v, idx_v, row_v):
        wid = lax.axis_index("s") * info.num_cores + lax.axis_index("c")

        def gather_column(tbl_hbm, out_hbm, f):
            pltpu.sync_copy(tbl_hbm.at[f], col_v)
            for h in range(2):
                pltpu.sync_copy(times_hbm.at[pl.ds(h * half, half)], idx_v)

                @plsc.parallel_loop(0, half, 16, unroll=8)
                def _(i):
                    iv = idx_v[pl.ds(i, 16)]
                    row_v[pl.ds(h * half + i, 16)] = (
                        plsc.load_gather(col_v, [iv]))

            pltpu.sync_copy(row_v, out_hbm.at[f])

        for fi in range(fpw):
            f = wid * fpw + fi
            gather_column(mean_hbm, mu_out, f)
            gather_column(std_hbm, sd_out, f)

    return run(times, mean_t, std_t)


def _tc_log_prob(states_t, mu_g, sd_g, b, d):
    blk = 2048
    grid = b // blk

    def body(st_ref, mu_ref, sd_ref, out_ref):
        s = st_ref[...]
        m = mu_ref[...]
        sig = jnp.maximum(sd_ref[...], 0.01)
        t = (s - m) / sig
        lp = -0.5 * (t * t) - jnp.log(sig)
        out_ref[0, 0, :] = jnp.sum(lp, axis=0) - (d * 0.5) * _LOG_2PI

    out = pl.pallas_call(
        body,
        grid=(grid,),
        in_specs=[
            pl.BlockSpec((d, blk), lambda i: (0, i)),
            pl.BlockSpec((d, blk), lambda i: (0, i)),
            pl.BlockSpec((d, blk), lambda i: (0, i)),
        ],
        out_specs=pl.BlockSpec((1, 1, blk), lambda i: (i, 0, 0)),
        out_shape=jax.ShapeDtypeStruct((grid, 1, blk), jnp.float32),
    )(states_t, mu_g, sd_g)
    return out.reshape(b)


def kernel(states, times, mean_params, std_params):
    b, d = states.shape
    n_times = mean_params.shape[0]
    times = times.reshape(-1).astype(jnp.int32)
    mu_g, sd_g = _sc_gather(times, mean_params.T, std_params.T, b, d, n_times)
    return _tc_log_prob(states.T, mu_g, sd_g, b, d)


# speedup vs baseline: 2.8354x; 1.0874x over previous
import functools

import jax
import jax.numpy as jnp
from jax import lax
from jax.experimental import pallas as pl
from jax.experimental.pallas import tpu as pltpu
from jax.experimental.pallas import tpu_sc as plsc

_LOG_2PI = 1.8378770664093453


def _sc_gather(times, mean_t, std_t, b, d, n_times):
    info = plsc.get_sparse_core_info()
    nw = info.num_cores * info.num_subcores
    fpw = d // nw
    q = 2048
    nq = b // q

    mesh = plsc.VectorSubcoreMesh(core_axis_name="c", subcore_axis_name="s")

    @functools.partial(
        pl.kernel,
        mesh=mesh,
        out_type=[
            jax.ShapeDtypeStruct((d, b), jnp.float32),
            jax.ShapeDtypeStruct((d, b), jnp.float32),
        ],
        scratch_types=[
            pltpu.VMEM((n_times,), jnp.float32),
            pltpu.VMEM((b,), jnp.int32),
            pltpu.VMEM((2, q), jnp.float32),
            pltpu.SemaphoreType.DMA,
            pltpu.SemaphoreType.DMA,
            pltpu.SemaphoreType.DMA,
        ],
        compiler_params=pltpu.CompilerParams(
            use_tc_tiling_on_sc=True,
            needs_layout_passes=False,
        ),
    )
    def run(times_hbm, mean_hbm, std_hbm, mu_out, sd_out,
            col_v, idx_v, st_v, csem, wsem0, wsem1):
        wid = lax.axis_index("s") * info.num_cores + lax.axis_index("c")
        passes = [(mean_hbm, mu_out, 0), (std_hbm, sd_out, 0),
                  (mean_hbm, mu_out, 1), (std_hbm, sd_out, 1)]
        wsems = [wsem0, wsem1]
        wr = [None, None]

        cp = pltpu.async_copy(mean_hbm.at[wid * fpw], col_v, csem)
        pltpu.sync_copy(times_hbm, idx_v)
        for p, (tbl, out, fi) in enumerate(passes):
            f = wid * fpw + fi
            cp.wait()
            for k in range(nq):
                sl = k & 1
                if wr[sl] is not None:
                    wr[sl].wait()

                @plsc.parallel_loop(0, q, 16, unroll=8)
                def _(i):
                    iv = idx_v[pl.ds(k * q + i, 16)]
                    st_v[sl, pl.ds(i, 16)] = plsc.load_gather(col_v, [iv])

                if k == nq - 1 and p + 1 < len(passes):
                    tbl2, _out2, fi2 = passes[p + 1]
                    cp = pltpu.async_copy(
                        tbl2.at[wid * fpw + fi2], col_v, csem)
                wr[sl] = pltpu.async_copy(
                    st_v.at[sl], out.at[f, pl.ds(k * q, q)], wsems[sl])
        wr[0].wait()
        wr[1].wait()

    return run(times, mean_t, std_t)


def _tc_log_prob(states_t, mu_g, sd_g, b, d):
    blk = 2048
    grid = b // blk

    def body(st_ref, mu_ref, sd_ref, out_ref):
        s = st_ref[...]
        m = mu_ref[...]
        sig = jnp.maximum(sd_ref[...], 0.01)
        t = (s - m) / sig
        lp = -0.5 * (t * t) - jnp.log(sig)
        out_ref[0, 0, :] = jnp.sum(lp, axis=0) - (d * 0.5) * _LOG_2PI

    out = pl.pallas_call(
        body,
        grid=(grid,),
        in_specs=[
            pl.BlockSpec((d, blk), lambda i: (0, i)),
            pl.BlockSpec((d, blk), lambda i: (0, i)),
            pl.BlockSpec((d, blk), lambda i: (0, i)),
        ],
        out_specs=pl.BlockSpec((1, 1, blk), lambda i: (i, 0, 0)),
        out_shape=jax.ShapeDtypeStruct((grid, 1, blk), jnp.float32),
    )(states_t, mu_g, sd_g)
    return out.reshape(b)


def kernel(states, times, mean_params, std_params):
    b, d = states.shape
    n_times = mean_params.shape[0]
    times = times.reshape(-1).astype(jnp.int32)
    mu_g, sd_g = _sc_gather(times, mean_params.T, std_params.T, b, d, n_times)
    return _tc_log_prob(states.T, mu_g, sd_g, b, d)
